# Initial kernel scaffold; baseline (speedup 1.0000x reference)
import jax
import jax.numpy as jnp
from jax.experimental import pallas as pl


def kernel(p, edge_index, W, b, raw_c):
    raise NotImplementedError("write your pallas kernel here")


# speedup vs baseline: 3.5169x; 3.5169x over previous
import functools
import math

import jax
import jax.numpy as jnp
from jax.experimental import pallas as pl
from jax.experimental.pallas import tpu as pltpu

N = 10000
E = 320000
D = 128
MIN_C = 1e-4
MAX_C = 10.0

ROW_BLK = 1000


def _pre_body(sc_ref, p_ref, w_ref, deg_ref, g_ref, uself_ref, dinv_ref):
    sqrt_c = sc_ref[0]
    p = p_ref[...]
    col = jax.lax.broadcasted_iota(jnp.int32, (ROW_BLK, D), 1)
    is_space = col > 0
    p0 = p[:, :1]
    ysq = jnp.sum(jnp.where(is_space, p * p, 0.0), axis=1, keepdims=True)
    yn = jnp.maximum(jnp.sqrt(ysq), 1e-12)
    x = jnp.maximum(sqrt_c * p0, 1.0 + 1e-7)
    r = jnp.log(x + jnp.sqrt(x * x - 1.0)) / sqrt_c
    v = jnp.where(is_space, p * (r / yn), 0.0)
    h = jnp.dot(v, w_ref[...], preferred_element_type=jnp.float32)
    dinv = jax.lax.rsqrt(deg_ref[...])
    g = dinv * h
    g_ref[...] = g
    uself_ref[...] = dinv * g
    dinv_ref[...] = dinv


def _post_body(sc_ref, s_ref, uself_ref, dinv_ref, b_ref, out_ref):
    sqrt_c = sc_ref[0]
    agg = dinv_ref[...] * s_ref[...] + uself_ref[...] + b_ref[...]
    w = jnp.maximum(agg, 0.0)
    col = jax.lax.broadcasted_iota(jnp.int32, (ROW_BLK, D), 1)
    is_space = col > 0
    usq = jnp.sum(jnp.where(is_space, w * w, 0.0), axis=1, keepdims=True)
    un = jnp.maximum(jnp.sqrt(usq), 1e-12)
    a = sqrt_c * un
    ea = jnp.exp(a)
    eia = 1.0 / ea
    time = (ea + eia) * (0.5 / sqrt_c)
    space_fac = (ea - eia) * 0.5 / a
    out_ref[...] = jnp.where(is_space, space_fac * w, time)


def _row_spec():
    return pl.BlockSpec((ROW_BLK, D), lambda i: (i, 0))


def _col_spec():
    return pl.BlockSpec((ROW_BLK, 1), lambda i: (i, 0))


def kernel(p, edge_index, W, b, raw_c):
    c = jnp.clip(jax.nn.softplus(raw_c), MIN_C, MAX_C)
    sqrt_c = jnp.sqrt(c).reshape((1,))
    src = edge_index[0]
    dst = edge_index[1]

    deg = (jnp.zeros((N,), jnp.float32).at[dst].add(1.0) + 1.0).reshape(N, 1)

    grid = N // ROW_BLK
    g, uself, dinv = pl.pallas_call(
        _pre_body,
        grid=(grid,),
        in_specs=[
            pl.BlockSpec(memory_space=pltpu.SMEM),
            _row_spec(),
            pl.BlockSpec((D, D), lambda i: (0, 0)),
            _col_spec(),
        ],
        out_specs=[_row_spec(), _row_spec(), _col_spec()],
        out_shape=[
            jax.ShapeDtypeStruct((N, D), jnp.float32),
            jax.ShapeDtypeStruct((N, D), jnp.float32),
            jax.ShapeDtypeStruct((N, 1), jnp.float32),
        ],
    )(sqrt_c, p, W, deg)

    s = jnp.zeros((N, D), jnp.float32).at[dst].add(g[src])

    out = pl.pallas_call(
        _post_body,
        grid=(grid,),
        in_specs=[
            pl.BlockSpec(memory_space=pltpu.SMEM),
            _row_spec(),
            _row_spec(),
            _col_spec(),
            pl.BlockSpec((1, D), lambda i: (0, 0)),
        ],
        out_specs=_row_spec(),
        out_shape=jax.ShapeDtypeStruct((N, D), jnp.float32),
    )(sqrt_c, s, uself, dinv, b.reshape(1, D))
    return out


# speedup vs baseline: 14.1884x; 4.0344x over previous
import functools
import math

import jax
import jax.numpy as jnp
from jax import lax
from jax.experimental import pallas as pl
from jax.experimental.pallas import tpu as pltpu
from jax.experimental.pallas import tpu_sc as plsc

N = 10000
E = 320000
D = 128
MIN_C = 1e-4
MAX_C = 10.0

NC = 2
NS = 16
NW = NC * NS

NPAD = 10240
DUMMY = N
BLK = 128
DH = D // 2
PERW = 10240
EP = PERW * NW
PERW2 = EP // NS
NB2 = PERW2 // BLK
TSLICE = NPAD // NS
ZROWS = NPAD // NS // BLK

ROW_BLK = 1024
OUT_BLK = 1000


def _mesh():
    return plsc.VectorSubcoreMesh(
        core_axis_name="c", subcore_axis_name="s",
        num_cores=NC, num_subcores=NS)


def _deg_body(dst_hbm, out_hbm, dstv, hist, tmp, acc, hists_sp):
    c = lax.axis_index("c")
    s = lax.axis_index("s")
    wid = c * NS + s
    pltpu.sync_copy(dst_hbm.at[pl.ds(wid * PERW, PERW)], dstv)

    zeros16 = jnp.zeros((16,), jnp.float32)
    ones16 = jnp.ones((16,), jnp.float32)

    def zero_body(i, _):
        hist[pl.ds(i * 16, 16)] = zeros16
        return 0
    lax.fori_loop(0, NPAD // 16, zero_body, 0)

    def hist_body(i, _):
        idx = dstv[pl.ds(i * 16, 16)]
        plsc.addupdate_scatter(hist, [idx], ones16)
        return 0
    lax.fori_loop(0, PERW // 16, hist_body, 0)

    pltpu.sync_copy(hist, hists_sp.at[s])
    plsc.subcore_barrier()

    base = s * TSLICE
    for j in range(NS):
        pltpu.sync_copy(hists_sp.at[j, pl.ds(base, TSLICE)], tmp.at[j])

    def red_body(k, _):
        v = tmp[0, pl.ds(k * 16, 16)]
        for j in range(1, NS):
            v = v + tmp[j, pl.ds(k * 16, 16)]
        acc[pl.ds(k * 16, 16)] = v
        return 0
    lax.fori_loop(0, TSLICE // 16, red_body, 0)

    pltpu.sync_copy(acc, out_hbm.at[c, pl.ds(base, TSLICE)])


def _deg_call(dst_flat):
    f = pl.kernel(
        _deg_body,
        out_type=jax.ShapeDtypeStruct((NC, NPAD), jnp.float32),
        mesh=_mesh(),
        compiler_params=pltpu.CompilerParams(needs_layout_passes=False),
        scratch_types=[
            pltpu.VMEM((PERW,), jnp.int32),
            pltpu.VMEM((NPAD,), jnp.float32),
            pltpu.VMEM((NS, TSLICE), jnp.float32),
            pltpu.VMEM((TSLICE,), jnp.float32),
            pltpu.VMEM_SHARED((NS, NPAD), jnp.float32),
        ],
    )
    return f(dst_flat)


def _edge_body(src_hbm, dst_hbm, g3_hbm, out_hbm,
               srcv, dstv, buf0, buf1, s_sp, sem0, sem1):
    c = lax.axis_index("c")
    s = lax.axis_index("s")
    pltpu.sync_copy(src_hbm.at[pl.ds(s * NB2, NB2)], srcv)
    pltpu.sync_copy(dst_hbm.at[pl.ds(s * NB2, NB2)], dstv)
    gh = g3_hbm.at[c]

    zeros16 = jnp.zeros((16,), jnp.float32)

    def zero_body(i, _):
        for k in range(DH // 16):
            buf0[i, pl.ds(k * 16, 16)] = zeros16
        return 0
    lax.fori_loop(0, BLK, zero_body, 0)

    base = s * (NPAD // NS)
    for j in range(ZROWS):
        pltpu.sync_copy(buf0, s_sp.at[pl.ds(base + j * BLK, BLK)])
    plsc.subcore_barrier()

    def body(k, _):
        j0 = 2 * k
        j1 = j0 + 1
        d0 = pltpu.async_copy(gh.at[srcv.at[j0]], buf0, sem0)
        d1 = pltpu.async_copy(gh.at[srcv.at[j1]], buf1, sem1)
        d0.wait()
        pltpu.sync_copy(buf0, s_sp.at[dstv.at[j0]], add=True)
        d1.wait()
        pltpu.sync_copy(buf1, s_sp.at[dstv.at[j1]], add=True)
        return 0
    lax.fori_loop(0, NB2 // 2, body, 0)

    plsc.subcore_barrier()
    rows = NPAD // NS
    pltpu.sync_copy(s_sp.at[pl.ds(base, rows)],
                    out_hbm.at[c, pl.ds(base, rows)])


def _edge_call(src2, dst2, g3):
    f = pl.kernel(
        _edge_body,
        out_type=jax.ShapeDtypeStruct((NC, NPAD, DH), jnp.float32),
        mesh=_mesh(),
        compiler_params=pltpu.CompilerParams(use_tc_tiling_on_sc=False),
        scratch_types=[
            pltpu.VMEM((NB2, BLK), jnp.int32),
            pltpu.VMEM((NB2, BLK), jnp.int32),
            pltpu.VMEM((BLK, DH), jnp.float32),
            pltpu.VMEM((BLK, DH), jnp.float32),
            pltpu.VMEM_SHARED((NPAD, DH), jnp.float32),
            pltpu.SemaphoreType.DMA,
            pltpu.SemaphoreType.DMA,
        ],
    )
    return f(src2, dst2, g3)


def _pre_body(sc_ref, p_ref, w_ref, d0_ref, d1_ref, g_ref, uself_ref, dinv_ref):
    sqrt_c = sc_ref[0]
    p = p_ref[...]
    col = lax.broadcasted_iota(jnp.int32, (ROW_BLK, D), 1)
    is_space = col > 0
    p0 = p[:, :1]
    ysq = jnp.sum(jnp.where(is_space, p * p, 0.0), axis=1, keepdims=True)
    yn = jnp.maximum(jnp.sqrt(ysq), 1e-12)
    x = jnp.maximum(sqrt_c * p0, 1.0 + 1e-7)
    r = jnp.log(x + jnp.sqrt(x * x - 1.0)) / sqrt_c
    v = jnp.where(is_space, p * (r / yn), 0.0)
    h = jnp.dot(v, w_ref[...], preferred_element_type=jnp.float32)
    dinv = lax.rsqrt(d0_ref[...] + d1_ref[...] + 1.0)
    g = dinv * h
    g_ref[0] = g[:, :DH]
    g_ref[1] = g[:, DH:]
    uself_ref[...] = dinv * g
    dinv_ref[...] = dinv


def _pre_call(sqrt_c, p, W, deg0, deg1):
    grid = NPAD // ROW_BLK
    row = pl.BlockSpec((ROW_BLK, D), lambda i: (i, 0))
    colb = pl.BlockSpec((ROW_BLK, 1), lambda i: (i, 0))
    return pl.pallas_call(
        _pre_body,
        grid=(grid,),
        in_specs=[
            pl.BlockSpec(memory_space=pltpu.SMEM),
            row,
            pl.BlockSpec((D, D), lambda i: (0, 0)),
            colb,
            colb,
        ],
        out_specs=[
            pl.BlockSpec((NC, ROW_BLK, DH), lambda i: (0, i, 0)),
            row,
            colb,
        ],
        out_shape=[
            jax.ShapeDtypeStruct((NC, NPAD, DH), jnp.float32),
            jax.ShapeDtypeStruct((NPAD, D), jnp.float32),
            jax.ShapeDtypeStruct((NPAD, 1), jnp.float32),
        ],
    )(sqrt_c, p, W, deg0, deg1)


def _post_body(sc_ref, s_ref, uself_ref, dinv_ref, b_ref, out_ref):
    sqrt_c = sc_ref[0]
    ssum = jnp.concatenate([s_ref[0], s_ref[1]], axis=1)
    agg = dinv_ref[...] * ssum + uself_ref[...] + b_ref[...]
    w = jnp.maximum(agg, 0.0)
    col = lax.broadcasted_iota(jnp.int32, (OUT_BLK, D), 1)
    is_space = col > 0
    usq = jnp.sum(jnp.where(is_space, w * w, 0.0), axis=1, keepdims=True)
    un = jnp.maximum(jnp.sqrt(usq), 1e-12)
    a = sqrt_c * un
    ea = jnp.exp(a)
    eia = 1.0 / ea
    time = (ea + eia) * (0.5 / sqrt_c)
    space_fac = (ea - eia) * 0.5 / a
    out_ref[...] = jnp.where(is_space, space_fac * w, time)


def _post_call(sqrt_c, s_pair, uself, dinv, b):
    grid = N // OUT_BLK
    row = pl.BlockSpec((OUT_BLK, D), lambda i: (i, 0))
    return pl.pallas_call(
        _post_body,
        grid=(grid,),
        in_specs=[
            pl.BlockSpec(memory_space=pltpu.SMEM),
            pl.BlockSpec((NC, OUT_BLK, DH), lambda i: (0, i, 0)),
            row,
            pl.BlockSpec((OUT_BLK, 1), lambda i: (i, 0)),
            pl.BlockSpec((1, D), lambda i: (0, 0)),
        ],
        out_specs=row,
        out_shape=jax.ShapeDtypeStruct((N, D), jnp.float32),
    )(sqrt_c, s_pair, uself, dinv, b.reshape(1, D))


def kernel(p, edge_index, W, b, raw_c):
    c = jnp.clip(jax.nn.softplus(raw_c), MIN_C, MAX_C)
    sqrt_c = jnp.sqrt(c).reshape((1,))
    pad = jnp.full((EP - E,), DUMMY, jnp.int32)
    src_flat = jnp.concatenate([edge_index[0], pad])
    dst_flat = jnp.concatenate([edge_index[1], pad])

    deg2 = _deg_call(dst_flat)
    deg0 = deg2[0].reshape(NPAD, 1)
    deg1 = deg2[1].reshape(NPAD, 1)

    g, uself, dinv = _pre_call(sqrt_c, p, W, deg0, deg1)

    s_pair = _edge_call(src_flat.reshape(EP // BLK, BLK),
                        dst_flat.reshape(EP // BLK, BLK), g)

    return _post_call(sqrt_c, s_pair, uself, dinv, b)


# speedup vs baseline: 16.5206x; 1.1644x over previous
import functools
import math

import jax
import jax.numpy as jnp
from jax import lax
from jax.experimental import pallas as pl
from jax.experimental.pallas import tpu as pltpu
from jax.experimental.pallas import tpu_sc as plsc

N = 10000
E = 320000
D = 128
MIN_C = 1e-4
MAX_C = 10.0

NC = 2
NS = 16
NW = NC * NS

NPAD = 10240
DUMMY = N
BLK = 128
DH = D // 2
PERW = 10240
EP = PERW * NW
PERW2 = EP // NS
NB2 = PERW2 // BLK
TSLICE = NPAD // NS
ZROWS = NPAD // NS // BLK

ROW_BLK = 1024
OUT_BLK = 1000


def _mesh():
    return plsc.VectorSubcoreMesh(
        core_axis_name="c", subcore_axis_name="s",
        num_cores=NC, num_subcores=NS)


def _deg_body(dst_hbm, out_hbm, dstv, hist, tmp, acc, hists_sp):
    c = lax.axis_index("c")
    s = lax.axis_index("s")
    wid = c * NS + s
    pltpu.sync_copy(dst_hbm.at[pl.ds(wid * PERW, PERW)], dstv)

    zeros16 = jnp.zeros((16,), jnp.float32)
    ones16 = jnp.ones((16,), jnp.float32)

    def zero_body(i, _):
        hist[pl.ds(i * 16, 16)] = zeros16
        return 0
    lax.fori_loop(0, NPAD // 16, zero_body, 0)

    def hist_body(i, _):
        idx = dstv[pl.ds(i * 16, 16)]
        plsc.addupdate_scatter(hist, [idx], ones16)
        return 0
    lax.fori_loop(0, PERW // 16, hist_body, 0)

    pltpu.sync_copy(hist, hists_sp.at[s])
    plsc.subcore_barrier()

    base = s * TSLICE
    for j in range(NS):
        pltpu.sync_copy(hists_sp.at[j, pl.ds(base, TSLICE)], tmp.at[j])

    def red_body(k, _):
        v = tmp[0, pl.ds(k * 16, 16)]
        for j in range(1, NS):
            v = v + tmp[j, pl.ds(k * 16, 16)]
        acc[pl.ds(k * 16, 16)] = v
        return 0
    lax.fori_loop(0, TSLICE // 16, red_body, 0)

    pltpu.sync_copy(acc, out_hbm.at[c, pl.ds(base, TSLICE)])


def _deg_call(dst_flat):
    f = pl.kernel(
        _deg_body,
        out_type=jax.ShapeDtypeStruct((NC, NPAD), jnp.float32),
        mesh=_mesh(),
        compiler_params=pltpu.CompilerParams(needs_layout_passes=False),
        scratch_types=[
            pltpu.VMEM((PERW,), jnp.int32),
            pltpu.VMEM((NPAD,), jnp.float32),
            pltpu.VMEM((NS, TSLICE), jnp.float32),
            pltpu.VMEM((TSLICE,), jnp.float32),
            pltpu.VMEM_SHARED((NS, NPAD), jnp.float32),
        ],
    )
    return f(dst_flat)


NBUF = 2
NGRP = NB2 // NBUF


def _edge_body(src_hbm, dst_hbm, g3_hbm, out_hbm,
               srcv, dstv, bufs, s_sp, sem_g, sem_s):
    c = lax.axis_index("c")
    s = lax.axis_index("s")
    pltpu.sync_copy(src_hbm.at[pl.ds(s * NB2, NB2)], srcv)
    pltpu.sync_copy(dst_hbm.at[pl.ds(s * NB2, NB2)], dstv)
    gh = g3_hbm.at[c]
    dummy = gh.at[pl.ds(0, BLK)]

    zeros16 = jnp.zeros((16,), jnp.float32)
    buf0 = bufs.at[0]

    def zero_body(i, _):
        for k in range(DH // 16):
            bufs[0, i, pl.ds(k * 16, 16)] = zeros16
        return 0
    lax.fori_loop(0, BLK, zero_body, 0)

    base = s * (NPAD // NS)
    for j in range(ZROWS):
        pltpu.sync_copy(buf0, s_sp.at[pl.ds(base + j * BLK, BLK)])
    plsc.subcore_barrier()

    for half in range(2):
        for b in range(NBUF):
            pltpu.async_copy(gh.at[srcv.at[half * NBUF + b]],
                             bufs.at[half * NBUF + b], sem_g)

    @pl.loop(0, NGRP, step=2)
    def outer(g):
        for half in range(2):
            gg = g + half
            bb = half * NBUF
            for b in range(NBUF):
                pltpu.make_async_copy(dummy, bufs.at[bb + b], sem_g).wait()
            for b in range(NBUF):
                pltpu.async_copy(bufs.at[bb + b],
                                 s_sp.at[dstv.at[gg * NBUF + b]], sem_s,
                                 add=True)
            for b in range(NBUF):
                pltpu.make_async_copy(dummy, bufs.at[bb + b], sem_s).wait()

            @pl.when(gg + 2 < NGRP)
            def _():
                for b in range(NBUF):
                    pltpu.async_copy(gh.at[srcv.at[(gg + 2) * NBUF + b]],
                                     bufs.at[bb + b], sem_g)

    plsc.subcore_barrier()
    rows = NPAD // NS
    pltpu.sync_copy(s_sp.at[pl.ds(base, rows)],
                    out_hbm.at[c, pl.ds(base, rows)])


def _edge_call(src2, dst2, g3):
    f = pl.kernel(
        _edge_body,
        out_type=jax.ShapeDtypeStruct((NC, NPAD, DH), jnp.float32),
        mesh=_mesh(),
        compiler_params=pltpu.CompilerParams(use_tc_tiling_on_sc=False),
        scratch_types=[
            pltpu.VMEM((NB2, BLK), jnp.int32),
            pltpu.VMEM((NB2, BLK), jnp.int32),
            pltpu.VMEM((2 * NBUF, BLK, DH), jnp.float32),
            pltpu.VMEM_SHARED((NPAD, DH), jnp.float32),
            pltpu.SemaphoreType.DMA,
            pltpu.SemaphoreType.DMA,
        ],
    )
    return f(src2, dst2, g3)


def _pre_body(sc_ref, p_ref, w_ref, d0_ref, d1_ref, g_ref, uself_ref, dinv_ref):
    sqrt_c = sc_ref[0]
    p = p_ref[...]
    col = lax.broadcasted_iota(jnp.int32, (ROW_BLK, D), 1)
    is_space = col > 0
    p0 = p[:, :1]
    ysq = jnp.sum(jnp.where(is_space, p * p, 0.0), axis=1, keepdims=True)
    yn = jnp.maximum(jnp.sqrt(ysq), 1e-12)
    x = jnp.maximum(sqrt_c * p0, 1.0 + 1e-7)
    r = jnp.log(x + jnp.sqrt(x * x - 1.0)) / sqrt_c
    v = jnp.where(is_space, p * (r / yn), 0.0)
    h = jnp.dot(v, w_ref[...], preferred_element_type=jnp.float32)
    dinv = lax.rsqrt(d0_ref[...] + d1_ref[...] + 1.0)
    g = dinv * h
    g_ref[0] = g[:, :DH]
    g_ref[1] = g[:, DH:]
    uself_ref[...] = dinv * g
    dinv_ref[...] = dinv


def _pre_call(sqrt_c, p, W, deg0, deg1):
    grid = NPAD // ROW_BLK
    row = pl.BlockSpec((ROW_BLK, D), lambda i: (i, 0))
    colb = pl.BlockSpec((ROW_BLK, 1), lambda i: (i, 0))
    return pl.pallas_call(
        _pre_body,
        grid=(grid,),
        in_specs=[
            pl.BlockSpec(memory_space=pltpu.SMEM),
            row,
            pl.BlockSpec((D, D), lambda i: (0, 0)),
            colb,
            colb,
        ],
        out_specs=[
            pl.BlockSpec((NC, ROW_BLK, DH), lambda i: (0, i, 0)),
            row,
            colb,
        ],
        out_shape=[
            jax.ShapeDtypeStruct((NC, NPAD, DH), jnp.float32),
            jax.ShapeDtypeStruct((NPAD, D), jnp.float32),
            jax.ShapeDtypeStruct((NPAD, 1), jnp.float32),
        ],
    )(sqrt_c, p, W, deg0, deg1)


def _post_body(sc_ref, s_ref, uself_ref, dinv_ref, b_ref, out_ref):
    sqrt_c = sc_ref[0]
    ssum = jnp.concatenate([s_ref[0], s_ref[1]], axis=1)
    agg = dinv_ref[...] * ssum + uself_ref[...] + b_ref[...]
    w = jnp.maximum(agg, 0.0)
    col = lax.broadcasted_iota(jnp.int32, (OUT_BLK, D), 1)
    is_space = col > 0
    usq = jnp.sum(jnp.where(is_space, w * w, 0.0), axis=1, keepdims=True)
    un = jnp.maximum(jnp.sqrt(usq), 1e-12)
    a = sqrt_c * un
    ea = jnp.exp(a)
    eia = 1.0 / ea
    time = (ea + eia) * (0.5 / sqrt_c)
    space_fac = (ea - eia) * 0.5 / a
    out_ref[...] = jnp.where(is_space, space_fac * w, time)


def _post_call(sqrt_c, s_pair, uself, dinv, b):
    grid = N // OUT_BLK
    row = pl.BlockSpec((OUT_BLK, D), lambda i: (i, 0))
    return pl.pallas_call(
        _post_body,
        grid=(grid,),
        in_specs=[
            pl.BlockSpec(memory_space=pltpu.SMEM),
            pl.BlockSpec((NC, OUT_BLK, DH), lambda i: (0, i, 0)),
            row,
            pl.BlockSpec((OUT_BLK, 1), lambda i: (i, 0)),
            pl.BlockSpec((1, D), lambda i: (0, 0)),
        ],
        out_specs=row,
        out_shape=jax.ShapeDtypeStruct((N, D), jnp.float32),
    )(sqrt_c, s_pair, uself, dinv, b.reshape(1, D))


def kernel(p, edge_index, W, b, raw_c):
    c = jnp.clip(jax.nn.softplus(raw_c), MIN_C, MAX_C)
    sqrt_c = jnp.sqrt(c).reshape((1,))
    pad = jnp.full((EP - E,), DUMMY, jnp.int32)
    src_flat = jnp.concatenate([edge_index[0], pad])
    dst_flat = jnp.concatenate([edge_index[1], pad])

    deg2 = _deg_call(dst_flat)
    deg0 = deg2[0].reshape(NPAD, 1)
    deg1 = deg2[1].reshape(NPAD, 1)

    g, uself, dinv = _pre_call(sqrt_c, p, W, deg0, deg1)

    s_pair = _edge_call(src_flat.reshape(EP // BLK, BLK),
                        dst_flat.reshape(EP // BLK, BLK), g)

    return _post_call(sqrt_c, s_pair, uself, dinv, b)
